# baseline (speedup 1.0000x reference)
import functools

import jax
import jax.numpy as jnp
from jax import lax
from jax.experimental import pallas as pl
from jax.experimental.pallas import tpu as pltpu
from jax.experimental.pallas import tpu_sc as plsc

MEMORY_SIZE = 100000
NGRAM_N = 4
NUM_HEADS = 4
HEAD_DIM = 128
EMBED_DIM = NUM_HEADS * HEAD_DIM
B, W, O = 1024, 50, 50
SEQ_LEN = O + W

_WRAP_MOD = (1 << 32) % MEMORY_SIZE


def _prime_table():
    ps = []
    base = 131
    for h in range(NUM_HEADS):
        x, r = base + h * 1009, []
        for _ in range(NGRAM_N):
            r.append(x)
            x = x * 31 + 1
        ps.append(r)
    return ps


_PRIMES = _prime_table()

NC, NS = 2, 16
NW = NC * NS
QTOT = B * W
QW = QTOT // NW
B_PER = B // NW
CQ = 16
RCH = CQ * NUM_HEADS
NCH = QW // CQ
NBUF = 5
PREF = 3


@functools.lru_cache(maxsize=None)
def _build_engram_sc():
    mesh = plsc.VectorSubcoreMesh(core_axis_name="c", subcore_axis_name="s")
    return functools.partial(
        pl.kernel,
        mesh=mesh,
        out_type=jax.ShapeDtypeStruct((QTOT, EMBED_DIM), jnp.float32),
        scratch_types=[
            pltpu.VMEM((B_PER, SEQ_LEN), jnp.int32),
            pltpu.VMEM((NCH, RCH), jnp.int32),
        ]
        + [pltpu.VMEM((RCH, HEAD_DIM), jnp.float32) for _ in range(NBUF)]
        + [pltpu.VMEM((CQ, EMBED_DIM), jnp.float32) for _ in range(NBUF)]
        + [pltpu.SemaphoreType.DMA for _ in range(2 * NBUF)],
        compiler_params=pltpu.CompilerParams(needs_layout_passes=False),
    )(_engram_sc)


def _engram_sc(seq_hbm, table_hbm, out_hbm, seq_v, idx_v, *bufs_sems):
    gbufs = bufs_sems[:NBUF]
    obufs = bufs_sems[NBUF : 2 * NBUF]
    gsems = bufs_sems[2 * NBUF : 3 * NBUF]
    wsems = bufs_sems[3 * NBUF :]

    wid = lax.axis_index("s") * NC + lax.axis_index("c")
    b0 = wid * B_PER
    wq0 = wid * QW

    pltpu.sync_copy(seq_hbm.at[pl.ds(b0, B_PER), :], seq_v)

    lanes = lax.iota(jnp.int32, 16)

    def hash_body(k, carry):
        qv = wq0 + k * 16 + lanes
        b = lax.div(qv, jnp.int32(W))
        w = qv - b * W
        brel = b - b0
        vals = []
        for i in range(NGRAM_N):
            col = w + (O - i)
            vals.append(plsc.load_gather(seq_v, [brel, col]))
        pos0 = (k * 16 + lanes) * NUM_HEADS
        for h in range(NUM_HEADS):
            hs = vals[0] * jnp.int32(_PRIMES[0][h])
            for i in range(1, NGRAM_N):
                hs = hs + vals[i] * jnp.int32(_PRIMES[i][h])
            m = lax.rem(hs, jnp.int32(MEMORY_SIZE))
            m = jnp.where(m < 0, m + MEMORY_SIZE, m)
            m = jnp.where(hs < 0, m + _WRAP_MOD, m)
            m = jnp.where(m >= MEMORY_SIZE, m - MEMORY_SIZE, m)
            fidx = m * NUM_HEADS + h
            pos = pos0 + h
            plsc.store_scatter(idx_v, [pos >> 6, pos & 63], fidx)
        return carry

    lax.fori_loop(0, QW // 16, hash_body, 0)

    def fire_gather(c, j):
        pltpu.async_copy(table_hbm.at[idx_v.at[c]], gbufs[j], gsems[j])

    def wait_gather(j):
        pltpu.make_async_copy(
            table_hbm.at[pl.ds(0, RCH), :], gbufs[j], gsems[j]
        ).wait()

    def fire_write(c, j):
        pltpu.async_copy(
            obufs[j], out_hbm.at[pl.ds(wq0 + c * CQ, CQ), :], wsems[j]
        )

    def wait_write(j):
        pltpu.make_async_copy(
            obufs[j], out_hbm.at[pl.ds(wq0, CQ), :], wsems[j]
        ).wait()

    for j in range(PREF):
        fire_gather(j, j)

    def repack(j):
        src = gbufs[j]
        dst = obufs[j]

        def rp_body(u, carry2):
            r = u * NUM_HEADS
            for h in range(NUM_HEADS):
                for v in range(HEAD_DIM // 16):
                    dst[u, pl.ds(h * HEAD_DIM + v * 16, 16)] = src[
                        r + h, pl.ds(v * 16, 16)
                    ]
            return carry2

        lax.fori_loop(0, CQ, rp_body, 0)

    def pipe_body(t, carry):
        for j in range(NBUF):
            c = t * NBUF + j
            wait_gather(j)

            @pl.when(t >= 1)
            def _():
                wait_write(j)

            repack(j)
            fire_write(c, j)

            jn = (j + PREF) % NBUF

            @pl.when(c + PREF < NCH)
            def _():
                fire_gather(c + PREF, jn)
        return carry

    lax.fori_loop(0, NCH // NBUF, pipe_body, 0)

    for j in range(NBUF):
        wait_write(j)


def _finish_body(g_ref, x_ref, o_ref):
    g = jax.nn.sigmoid(g_ref[...])
    x = x_ref[...]
    o_ref[...] = x.reshape(8, W, EMBED_DIM) * g.reshape(1, 1, EMBED_DIM)


@functools.lru_cache(maxsize=None)
def _build_finish_tc():
    return pl.pallas_call(
        _finish_body,
        grid=(B // 8,),
        in_specs=[
            pl.BlockSpec((1, EMBED_DIM), lambda i: (0, 0)),
            pl.BlockSpec((8 * W, EMBED_DIM), lambda i: (i, 0)),
        ],
        out_specs=pl.BlockSpec((8, W, EMBED_DIM), lambda i: (i, 0, 0)),
        out_shape=jax.ShapeDtypeStruct((B, W, EMBED_DIM), jnp.float32),
    )


@jax.jit
def kernel(curr, prev, table, gate):
    seq = jnp.concatenate([prev, curr], axis=1)
    table_flat = table.reshape(MEMORY_SIZE * NUM_HEADS, HEAD_DIM)
    inter = _build_engram_sc()(seq, table_flat)
    return _build_finish_tc()(gate.reshape(1, EMBED_DIM), inter)


# speedup vs baseline: 1.1652x; 1.1652x over previous
import functools

import jax
import jax.numpy as jnp
from jax import lax
from jax.experimental import pallas as pl
from jax.experimental.pallas import tpu as pltpu
from jax.experimental.pallas import tpu_sc as plsc

MEMORY_SIZE = 100000
NGRAM_N = 4
NUM_HEADS = 4
HEAD_DIM = 128
EMBED_DIM = NUM_HEADS * HEAD_DIM
B, W, O = 1024, 50, 50
SEQ_LEN = O + W

_WRAP_MOD = (1 << 32) % MEMORY_SIZE


def _prime_table():
    ps = []
    base = 131
    for h in range(NUM_HEADS):
        x, r = base + h * 1009, []
        for _ in range(NGRAM_N):
            r.append(x)
            x = x * 31 + 1
        ps.append(r)
    return ps


_PRIMES = _prime_table()

NC, NS = 2, 16
NW = NC * NS
QTOT = B * W
QW = QTOT // NW
B_PER = B // NW
RCH = W * NUM_HEADS
HROW = 100
WT = W // 8
WREM = W - 8 * WT


@functools.lru_cache(maxsize=None)
def _build_engram_sc():
    mesh = plsc.VectorSubcoreMesh(core_axis_name="c", subcore_axis_name="s")
    return functools.partial(
        pl.kernel,
        mesh=mesh,
        out_type=jax.ShapeDtypeStruct((B, W, EMBED_DIM), jnp.float32),
        scratch_types=[
            pltpu.VMEM((B_PER, SEQ_LEN), jnp.int32),
            pltpu.VMEM((NUM_HEADS, HEAD_DIM), jnp.float32),
            pltpu.VMEM((2 * B_PER, HROW), jnp.int32),
            pltpu.VMEM((RCH, HEAD_DIM), jnp.float32),
            pltpu.VMEM((RCH, HEAD_DIM), jnp.float32),
            pltpu.VMEM((W, EMBED_DIM), jnp.float32),
            pltpu.VMEM((W, EMBED_DIM), jnp.float32),
            pltpu.SemaphoreType.DMA,
            pltpu.SemaphoreType.DMA,
            pltpu.SemaphoreType.DMA,
            pltpu.SemaphoreType.DMA,
        ],
        compiler_params=pltpu.CompilerParams(needs_layout_passes=False),
    )(_engram_sc)


def _engram_sc(
    seq_hbm,
    table_hbm,
    gate_hbm,
    out_hbm,
    seq_v,
    g_v,
    idx_v,
    ga0,
    ga1,
    ob0,
    ob1,
    gsem0,
    gsem1,
    wsem0,
    wsem1,
):
    gbufs = (ga0, ga1)
    obufs = (ob0, ob1)
    gsems = (gsem0, gsem1)
    wsems = (wsem0, wsem1)

    wid = lax.axis_index("s") * NC + lax.axis_index("c")
    b0 = wid * B_PER
    wq0 = wid * QW

    pltpu.sync_copy(seq_hbm.at[pl.ds(b0, B_PER), :], seq_v)
    pltpu.sync_copy(gate_hbm, g_v)
    for h in range(NUM_HEADS):
        for v in range(HEAD_DIM // 16):
            sl = pl.ds(v * 16, 16)
            x = g_v[h, sl]
            g_v[h, sl] = 1.0 / (1.0 + jnp.exp(-x))

    lanes = lax.iota(jnp.int32, 16)

    def hash_body(k, carry):
        qv = wq0 + k * 16 + lanes
        b = lax.div(qv, jnp.int32(W))
        w = qv - b * W
        brel = b - b0
        vals = []
        for i in range(NGRAM_N):
            col = w + (O - i)
            vals.append(plsc.load_gather(seq_v, [brel, col]))
        pos0 = (k * 16 + lanes) * NUM_HEADS
        for h in range(NUM_HEADS):
            hs = vals[0] * jnp.int32(_PRIMES[0][h])
            for i in range(1, NGRAM_N):
                hs = hs + vals[i] * jnp.int32(_PRIMES[i][h])
            m = lax.rem(hs, jnp.int32(MEMORY_SIZE))
            m = jnp.where(m < 0, m + MEMORY_SIZE, m)
            m = jnp.where(hs < 0, m + _WRAP_MOD, m)
            m = jnp.where(m >= MEMORY_SIZE, m - MEMORY_SIZE, m)
            fidx = m * NUM_HEADS + h
            pos = pos0 + h
            prow = lax.div(pos, jnp.int32(HROW))
            plsc.store_scatter(idx_v, [prow, pos - prow * HROW], fidx)
        return carry

    lax.fori_loop(0, QW // 16, hash_body, 0)

    def fire_gather(c, j):
        pltpu.async_copy(
            table_hbm.at[idx_v.at[2 * c]], gbufs[j].at[pl.ds(0, HROW), :], gsems[j]
        )
        pltpu.async_copy(
            table_hbm.at[idx_v.at[2 * c + 1]],
            gbufs[j].at[pl.ds(HROW, HROW), :],
            gsems[j],
        )

    def wait_gather(j):
        pltpu.make_async_copy(
            table_hbm.at[pl.ds(0, RCH), :], gbufs[j], gsems[j]
        ).wait()

    def fire_write(c, j):
        bb = b0 + c
        for t in range(WT):
            pltpu.async_copy(
                obufs[j].at[pl.ds(8 * t, 8), :],
                out_hbm.at[bb, pl.ds(8 * t, 8), :],
                wsems[j],
            )
        pltpu.async_copy(
            obufs[j].at[pl.ds(8 * WT, WREM), :],
            out_hbm.at[bb, pl.ds(8 * WT, WREM), :],
            wsems[j],
        )

    def wait_write(j):
        pltpu.make_async_copy(obufs[j], out_hbm.at[b0], wsems[j]).wait()

    gv = [
        [g_v[h, pl.ds(v * 16, 16)] for v in range(HEAD_DIM // 16)]
        for h in range(NUM_HEADS)
    ]

    def scale_repack(j):
        src = gbufs[j]
        dst = obufs[j]

        def rp_body(u, carry2):
            r = u * NUM_HEADS
            for h in range(NUM_HEADS):
                for v in range(HEAD_DIM // 16):
                    dst[u, pl.ds(h * HEAD_DIM + v * 16, 16)] = (
                        src[r + h, pl.ds(v * 16, 16)] * gv[h][v]
                    )
            return carry2

        lax.fori_loop(0, W, rp_body, 0)

    fire_gather(0, 0)

    def pipe_body(t, carry):
        for jj in range(2):
            c = t * 2 + jj

            @pl.when(c + 1 < B_PER)
            def _():
                fire_gather(c + 1, 1 - jj)

            wait_gather(jj)

            @pl.when(c >= 2)
            def _():
                wait_write(jj)

            scale_repack(jj)
            fire_write(c, jj)
        return carry

    lax.fori_loop(0, B_PER // 2, pipe_body, 0)

    wait_write(0)
    wait_write(1)


@jax.jit
def kernel(curr, prev, table, gate):
    seq = jnp.concatenate([prev, curr], axis=1)
    table_flat = table.reshape(MEMORY_SIZE * NUM_HEADS, HEAD_DIM)
    return _build_engram_sc()(seq, table_flat, gate)
